# Initial kernel scaffold; baseline (speedup 1.0000x reference)
import functools

import jax
import jax.numpy as jnp
from jax import lax
from jax.experimental import pallas as pl
from jax.experimental.pallas import tpu as pltpu
from jax.experimental.pallas import tpu_sc as plsc

N = 10000
E = 320000
D = 128
ED = 16
H1 = 5
H2 = 1
G = 64

NPAD = 10240
NW = 32
EW = E // NW
LANES = 16
HIGHEST = lax.Precision.HIGHEST


def _tc_node_proj(w_ref, b_ref, x_ref, out_ref):
    out = lax.dot_general(w_ref[...], x_ref[...], (((0,), (1,)), ((), ())),
                          preferred_element_type=jnp.float32, precision=HIGHEST)
    out_ref[...] = out + b_ref[...]


def _node_proj(x_pad, Wcat, bcat):
    return pl.pallas_call(
        _tc_node_proj,
        out_shape=jax.ShapeDtypeStruct((24, NPAD), jnp.float32),
    )(Wcat, bcat, x_pad)


BLK_E = 2560


def _tc_edge_proj(w_ref, b_ref, ea_ref, out_ref):
    out = lax.dot_general(w_ref[...], ea_ref[...], (((0,), (1,)), ((), ())),
                          preferred_element_type=jnp.float32, precision=HIGHEST)
    out_ref[...] = out + b_ref[...]


def _edge_proj(Wecat, becat, edge_attr):
    grid = (E // BLK_E,)
    return pl.pallas_call(
        _tc_edge_proj,
        grid=grid,
        in_specs=[
            pl.BlockSpec((ED, 8), lambda i: (0, 0)),
            pl.BlockSpec((8, 1), lambda i: (0, 0)),
            pl.BlockSpec((BLK_E, ED), lambda i: (i, 0)),
        ],
        out_specs=pl.BlockSpec((8, BLK_E), lambda i: (0, i)),
        out_shape=jax.ShapeDtypeStruct((8, E), jnp.float32),
    )(Wecat, becat, edge_attr)


_MESH = plsc.VectorSubcoreMesh(core_axis_name="c", subcore_axis_name="s")


def _sc_edge_pass(nch, ch_rows, e_row0, kqv_hbm, e_hbm, ei_hbm, out_hbm,
                  src_v, dst_v, ktab, qtab, vtab, e_v, agg):
    wid = lax.axis_index("s") * 2 + lax.axis_index("c")
    base = wid * EW
    pltpu.sync_copy(ei_hbm.at[0, pl.ds(base, EW)], src_v)
    pltpu.sync_copy(ei_hbm.at[1, pl.ds(base, EW)], dst_v)
    kr, qr, vr = ch_rows
    zero = jnp.zeros((LANES,), jnp.float32)
    for c in range(nch):
        pltpu.sync_copy(kqv_hbm.at[kr + c], ktab)
        pltpu.sync_copy(kqv_hbm.at[qr + c], qtab)
        pltpu.sync_copy(kqv_hbm.at[vr + c], vtab)
        pltpu.sync_copy(e_hbm.at[e_row0 + c, pl.ds(base, EW)], e_v)

        def zbody(i, _):
            agg[pl.ds(i * LANES, LANES)] = zero
            return 0
        lax.fori_loop(0, NPAD // LANES, zbody, 0)

        def ebody(i, _):
            off = i * LANES
            sidx = src_v[pl.ds(off, LANES)]
            didx = dst_v[pl.ds(off, LANES)]
            kd = plsc.load_gather(ktab, [didx])
            qs = plsc.load_gather(qtab, [sidx])
            vs = plsc.load_gather(vtab, [sidx])
            ev = e_v[pl.ds(off, LANES)]
            z = kd + qs + ev
            g = 1.0 / (1.0 + jnp.exp(-z))
            plsc.addupdate_scatter(agg, [didx], g * vs)
            return 0
        lax.fori_loop(0, EW // LANES, ebody, 0)

        if nch == 1:
            pltpu.sync_copy(agg, out_hbm.at[wid])
        else:
            pltpu.sync_copy(agg, out_hbm.at[c, wid])


def _sc_scratch():
    return [
        pltpu.VMEM((EW,), jnp.int32),
        pltpu.VMEM((EW,), jnp.int32),
        pltpu.VMEM((NPAD,), jnp.float32),
        pltpu.VMEM((NPAD,), jnp.float32),
        pltpu.VMEM((NPAD,), jnp.float32),
        pltpu.VMEM((EW,), jnp.float32),
        pltpu.VMEM((NPAD,), jnp.float32),
    ]


@functools.partial(
    pl.kernel,
    out_type=jax.ShapeDtypeStruct((H1, NW, NPAD), jnp.float32),
    mesh=_MESH,
    scratch_types=_sc_scratch(),
)
def _sc_conv1(kqv_hbm, e_hbm, ei_hbm, out_hbm, *scratch):
    _sc_edge_pass(H1, (0, 5, 10), 0, kqv_hbm, e_hbm, ei_hbm, out_hbm, *scratch)


@functools.partial(
    pl.kernel,
    out_type=jax.ShapeDtypeStruct((NW, NPAD), jnp.float32),
    mesh=_MESH,
    scratch_types=_sc_scratch(),
)
def _sc_conv2(kqv_hbm, e_hbm, ei_hbm, out_hbm, *scratch):
    _sc_edge_pass(1, (0, 1, 2), 5, kqv_hbm, e_hbm, ei_hbm, out_hbm, *scratch)


def _tc_norm(parts_ref, kqv_ref, batch_ref, gw_ref, gb_ref, gms_ref,
             w2_ref, b2_ref, out_ref):
    rows = parts_ref[...]
    hs = []
    for c in range(H1):
        agg_c = jnp.sum(rows[c * NW:(c + 1) * NW, :], axis=0, keepdims=True)
        hs.append(agg_c + kqv_ref[15 + c:16 + c, :])
    h = jnp.concatenate(hs, axis=0)

    bcol = batch_ref[...]
    seg = lax.broadcasted_iota(jnp.int32, (NPAD, G), 1)
    m = jnp.where(bcol == seg, 1.0, 0.0)
    cnt = jnp.maximum(jnp.sum(m, axis=0, keepdims=True), 1.0)
    sums = lax.dot_general(h, m, (((1,), (0,)), ((), ())),
                           preferred_element_type=jnp.float32, precision=HIGHEST)
    mean = sums / cnt
    mean_x = lax.dot_general(mean, m, (((1,), (1,)), ((), ())),
                             preferred_element_type=jnp.float32, precision=HIGHEST)
    cent = h - gms_ref[...] * mean_x
    var = lax.dot_general(cent * cent, m, (((1,), (0,)), ((), ())),
                          preferred_element_type=jnp.float32, precision=HIGHEST) / cnt
    std = jnp.sqrt(var + 1e-5)
    std_x = lax.dot_general(std, m, (((1,), (1,)), ((), ())),
                            preferred_element_type=jnp.float32, precision=HIGHEST)
    normed = gw_ref[...] * cent / std_x + gb_ref[...]
    h2 = jnp.maximum(normed, 0.0)
    rows4 = lax.dot_general(w2_ref[...], h2, (((0,), (0,)), ((), ())),
                            preferred_element_type=jnp.float32, precision=HIGHEST)
    out_ref[...] = rows4 + b2_ref[...]


def _norm_stage(parts1_2d, kqv_cm, batch2d, gw, gb, gms, W2cat, b2cat):
    return pl.pallas_call(
        _tc_norm,
        out_shape=jax.ShapeDtypeStruct((4, NPAD), jnp.float32),
    )(parts1_2d, kqv_cm, batch2d, gw, gb, gms, W2cat, b2cat)


def _tc_final(parts_ref, kqvs_ref, out_ref):
    s = jnp.sum(parts_ref[...], axis=0, keepdims=True)
    z = s + kqvs_ref[3:4, :]
    out_ref[...] = 1.0 / (1.0 + jnp.exp(-z))


def _final_stage(parts2, kqvs2):
    return pl.pallas_call(
        _tc_final,
        out_shape=jax.ShapeDtypeStruct((1, NPAD), jnp.float32),
    )(parts2, kqvs2)


def kernel(x, edge_index, edge_attr, batch_idx, Wk1, bk1, Wq1, bq1, Wv1, bv1,
           We1, be1, Ws1, b1, gw, gb, gms, Wk2, bk2, Wq2, bq2, Wv2, bv2,
           We2, be2, Ws2, b2):
    x_pad = jnp.pad(x, ((0, NPAD - N), (0, 0)))
    Wcat = jnp.concatenate(
        [Wk1, Wq1, Wv1, Ws1, jnp.zeros((D, 4), jnp.float32)], axis=1)
    bcat = jnp.concatenate(
        [bk1, bq1, bv1, b1, jnp.zeros((4,), jnp.float32)])[:, None]
    kqv_cm = _node_proj(x_pad, Wcat, bcat)

    Wecat = jnp.concatenate(
        [We1, We2, jnp.zeros((ED, 2), jnp.float32)], axis=1)
    becat = jnp.concatenate(
        [be1, be2, jnp.zeros((2,), jnp.float32)])[:, None]
    e_cm = _edge_proj(Wecat, becat, edge_attr)

    parts1 = _sc_conv1(kqv_cm, e_cm, edge_index)

    batch2d = jnp.pad(batch_idx, (0, NPAD - N), constant_values=G)[:, None]
    W2cat = jnp.concatenate([Wk2, Wq2, Wv2, Ws2], axis=1)
    b2cat = jnp.concatenate([bk2, bq2, bv2, b2])[:, None]
    kqvs2 = _norm_stage(parts1.reshape(H1 * NW, NPAD), kqv_cm, batch2d,
                        gw[:, None], gb[:, None], gms[:, None], W2cat, b2cat)

    parts2 = _sc_conv2(kqvs2, e_cm, edge_index)
    out = _final_stage(parts2, kqvs2)
    return out[0, :N].reshape(N, 1)


# speedup vs baseline: 21.2632x; 21.2632x over previous
import functools

import jax
import jax.numpy as jnp
from jax import lax
from jax.experimental import pallas as pl
from jax.experimental.pallas import tpu as pltpu
from jax.experimental.pallas import tpu_sc as plsc

N = 10000
E = 320000
D = 128
ED = 16
H1 = 5
H2 = 1
G = 64

NPAD = 10240
NW = 32
EW = E // NW
LANES = 16
HIGHEST = lax.Precision.HIGHEST


def _tc_node_proj(w_ref, b_ref, x_ref, out_ref):
    out = lax.dot_general(w_ref[...], x_ref[...], (((0,), (1,)), ((), ())),
                          preferred_element_type=jnp.float32, precision=HIGHEST)
    out_ref[...] = out + b_ref[...]


def _node_proj(x_pad, Wcat, bcat):
    return pl.pallas_call(
        _tc_node_proj,
        out_shape=jax.ShapeDtypeStruct((24, NPAD), jnp.float32),
    )(Wcat, bcat, x_pad)


BLK_E = 2560


def _tc_edge_proj(w_ref, b_ref, ea_ref, out_ref):
    out = lax.dot_general(w_ref[...], ea_ref[...], (((0,), (1,)), ((), ())),
                          preferred_element_type=jnp.float32, precision=HIGHEST)
    out_ref[...] = out + b_ref[...]


def _edge_proj(Wecat, becat, edge_attr):
    grid = (E // BLK_E,)
    return pl.pallas_call(
        _tc_edge_proj,
        grid=grid,
        in_specs=[
            pl.BlockSpec((ED, 8), lambda i: (0, 0)),
            pl.BlockSpec((8, 1), lambda i: (0, 0)),
            pl.BlockSpec((BLK_E, ED), lambda i: (i, 0)),
        ],
        out_specs=pl.BlockSpec((8, BLK_E), lambda i: (0, i)),
        out_shape=jax.ShapeDtypeStruct((8, E), jnp.float32),
    )(Wecat, becat, edge_attr)


@functools.lru_cache(maxsize=None)
def _sc_mesh():
    return plsc.VectorSubcoreMesh(core_axis_name="c", subcore_axis_name="s",
                                  num_cores=2, num_subcores=16)


def _sc_edge_pass(nch, ch_rows, e_row0, kqv_hbm, e_hbm, ei_hbm, out_hbm,
                  src_v, dst_v, ktab, qtab, vtab, e_v, agg):
    wid = lax.axis_index("s") * 2 + lax.axis_index("c")
    base = wid * EW
    pltpu.sync_copy(ei_hbm.at[pl.ds(base, EW)], src_v)
    pltpu.sync_copy(ei_hbm.at[pl.ds(E + base, EW)], dst_v)
    kr, qr, vr = ch_rows
    zero = jnp.zeros((LANES,), jnp.float32)
    for c in range(nch):
        pltpu.sync_copy(kqv_hbm.at[pl.ds((kr + c) * NPAD, NPAD)], ktab)
        pltpu.sync_copy(kqv_hbm.at[pl.ds((qr + c) * NPAD, NPAD)], qtab)
        pltpu.sync_copy(kqv_hbm.at[pl.ds((vr + c) * NPAD, NPAD)], vtab)
        pltpu.sync_copy(e_hbm.at[pl.ds((e_row0 + c) * E + base, EW)], e_v)

        def zbody(i, _):
            agg[pl.ds(i * LANES, LANES)] = zero
            return 0
        lax.fori_loop(0, NPAD // LANES, zbody, 0)

        def ebody(i, _):
            off = i * LANES
            sidx = src_v[pl.ds(off, LANES)]
            didx = dst_v[pl.ds(off, LANES)]
            kd = plsc.load_gather(ktab, [didx])
            qs = plsc.load_gather(qtab, [sidx])
            vs = plsc.load_gather(vtab, [sidx])
            ev = e_v[pl.ds(off, LANES)]
            z = kd + qs + ev
            g = 1.0 / (1.0 + jnp.exp(-z))
            plsc.addupdate_scatter(agg, [didx], g * vs)
            return 0
        lax.fori_loop(0, EW // LANES, ebody, 0)

        pltpu.sync_copy(agg, out_hbm.at[pl.ds((c * NW + wid) * NPAD, NPAD)])


def _sc_scratch():
    return [
        pltpu.VMEM((EW,), jnp.int32),
        pltpu.VMEM((EW,), jnp.int32),
        pltpu.VMEM((NPAD,), jnp.float32),
        pltpu.VMEM((NPAD,), jnp.float32),
        pltpu.VMEM((NPAD,), jnp.float32),
        pltpu.VMEM((EW,), jnp.float32),
        pltpu.VMEM((NPAD,), jnp.float32),
    ]


@functools.lru_cache(maxsize=None)
def _sc_conv1():
    @functools.partial(
        pl.kernel,
        out_type=jax.ShapeDtypeStruct((H1 * NW * NPAD,), jnp.float32),
        mesh=_sc_mesh(),
        scratch_types=_sc_scratch(),
        compiler_params=pltpu.CompilerParams(needs_layout_passes=False),
    )
    def body(kqv_hbm, e_hbm, ei_hbm, out_hbm, *scratch):
        _sc_edge_pass(H1, (0, 5, 10), 0, kqv_hbm, e_hbm, ei_hbm, out_hbm,
                      *scratch)
    return body


@functools.lru_cache(maxsize=None)
def _sc_conv2():
    @functools.partial(
        pl.kernel,
        out_type=jax.ShapeDtypeStruct((NW * NPAD,), jnp.float32),
        mesh=_sc_mesh(),
        scratch_types=_sc_scratch(),
        compiler_params=pltpu.CompilerParams(needs_layout_passes=False),
    )
    def body(kqv_hbm, e_hbm, ei_hbm, out_hbm, *scratch):
        _sc_edge_pass(1, (0, 1, 2), 5, kqv_hbm, e_hbm, ei_hbm, out_hbm,
                      *scratch)
    return body


def _tc_norm(parts_ref, kqv_ref, batch_ref, gw_ref, gb_ref, gms_ref,
             w2_ref, b2_ref, out_ref):
    rows = parts_ref[...]
    hs = []
    for c in range(H1):
        agg_c = jnp.sum(rows[c * NW:(c + 1) * NW, :], axis=0, keepdims=True)
        hs.append(agg_c + kqv_ref[15 + c:16 + c, :])
    h = jnp.concatenate(hs, axis=0)

    bcol = batch_ref[...]
    seg = lax.broadcasted_iota(jnp.int32, (NPAD, G), 1)
    m = jnp.where(bcol == seg, 1.0, 0.0)
    cnt = jnp.maximum(jnp.sum(m, axis=0, keepdims=True), 1.0)
    sums = lax.dot_general(h, m, (((1,), (0,)), ((), ())),
                           preferred_element_type=jnp.float32, precision=HIGHEST)
    mean = sums / cnt
    mean_x = lax.dot_general(mean, m, (((1,), (1,)), ((), ())),
                             preferred_element_type=jnp.float32, precision=HIGHEST)
    cent = h - gms_ref[...] * mean_x
    var = lax.dot_general(cent * cent, m, (((1,), (0,)), ((), ())),
                          preferred_element_type=jnp.float32, precision=HIGHEST) / cnt
    std = jnp.sqrt(var + 1e-5)
    std_x = lax.dot_general(std, m, (((1,), (1,)), ((), ())),
                            preferred_element_type=jnp.float32, precision=HIGHEST)
    normed = gw_ref[...] * cent / std_x + gb_ref[...]
    h2 = jnp.maximum(normed, 0.0)
    rows4 = lax.dot_general(w2_ref[...], h2, (((0,), (0,)), ((), ())),
                            preferred_element_type=jnp.float32, precision=HIGHEST)
    out_ref[...] = rows4 + b2_ref[...]


def _norm_stage(parts1_2d, kqv_cm, batch2d, gw, gb, gms, W2cat, b2cat):
    return pl.pallas_call(
        _tc_norm,
        out_shape=jax.ShapeDtypeStruct((4, NPAD), jnp.float32),
    )(parts1_2d, kqv_cm, batch2d, gw, gb, gms, W2cat, b2cat)


def _tc_final(parts_ref, kqvs_ref, out_ref):
    s = jnp.sum(parts_ref[...], axis=0, keepdims=True)
    z = s + kqvs_ref[3:4, :]
    out_ref[...] = 1.0 / (1.0 + jnp.exp(-z))


def _final_stage(parts2, kqvs2):
    return pl.pallas_call(
        _tc_final,
        out_shape=jax.ShapeDtypeStruct((1, NPAD), jnp.float32),
    )(parts2, kqvs2)


def kernel(x, edge_index, edge_attr, batch_idx, Wk1, bk1, Wq1, bq1, Wv1, bv1,
           We1, be1, Ws1, b1, gw, gb, gms, Wk2, bk2, Wq2, bq2, Wv2, bv2,
           We2, be2, Ws2, b2):
    x_pad = jnp.pad(x, ((0, NPAD - N), (0, 0)))
    Wcat = jnp.concatenate(
        [Wk1, Wq1, Wv1, Ws1, jnp.zeros((D, 4), jnp.float32)], axis=1)
    bcat = jnp.concatenate(
        [bk1, bq1, bv1, b1, jnp.zeros((4,), jnp.float32)])[:, None]
    kqv_cm = _node_proj(x_pad, Wcat, bcat)

    Wecat = jnp.concatenate(
        [We1, We2, jnp.zeros((ED, 2), jnp.float32)], axis=1)
    becat = jnp.concatenate(
        [be1, be2, jnp.zeros((2,), jnp.float32)])[:, None]
    e_cm = _edge_proj(Wecat, becat, edge_attr)

    kqv_flat = kqv_cm.reshape(-1)
    e_flat = e_cm.reshape(-1)
    ei_flat = edge_index.reshape(-1)
    parts1 = _sc_conv1()(kqv_flat, e_flat, ei_flat)

    batch2d = jnp.pad(batch_idx, (0, NPAD - N), constant_values=G)[:, None]
    W2cat = jnp.concatenate([Wk2, Wq2, Wv2, Ws2], axis=1)
    b2cat = jnp.concatenate([bk2, bq2, bv2, b2])[:, None]
    kqvs2 = _norm_stage(parts1.reshape(H1 * NW, NPAD), kqv_cm, batch2d,
                        gw[:, None], gb[:, None], gms[:, None], W2cat, b2cat)

    parts2 = _sc_conv2()(kqvs2.reshape(-1), e_flat, ei_flat)
    out = _final_stage(parts2.reshape(NW, NPAD), kqvs2)
    return out[0, :N].reshape(N, 1)


# speedup vs baseline: 32.4133x; 1.5244x over previous
import functools

import jax
import jax.numpy as jnp
from jax import lax
from jax.experimental import pallas as pl
from jax.experimental.pallas import tpu as pltpu
from jax.experimental.pallas import tpu_sc as plsc

N = 10000
E = 320000
D = 128
ED = 16
H1 = 5
H2 = 1
G = 64

NPAD = 10240
NW = 32
EW = E // NW
LANES = 16
HIGHEST = lax.Precision.HIGHEST


def _tc_node_proj(w_ref, b_ref, x_ref, out_ref):
    out = lax.dot_general(w_ref[...], x_ref[...], (((0,), (1,)), ((), ())),
                          preferred_element_type=jnp.float32, precision=HIGHEST)
    out_ref[...] = out + b_ref[...]


def _node_proj(x_pad, Wcat, bcat):
    return pl.pallas_call(
        _tc_node_proj,
        out_shape=jax.ShapeDtypeStruct((24, NPAD), jnp.float32),
    )(Wcat, bcat, x_pad)


BLK_E = 2560


def _tc_edge_proj(w_ref, b_ref, ea_ref, out_ref):
    out = lax.dot_general(w_ref[...], ea_ref[...], (((0,), (0,)), ((), ())),
                          preferred_element_type=jnp.float32, precision=HIGHEST)
    out_ref[...] = out + b_ref[...]


def _edge_proj(Wecat, becat, ea_t):
    grid = (E // BLK_E,)
    return pl.pallas_call(
        _tc_edge_proj,
        grid=grid,
        in_specs=[
            pl.BlockSpec((ED, 8), lambda i: (0, 0)),
            pl.BlockSpec((8, 1), lambda i: (0, 0)),
            pl.BlockSpec((ED, BLK_E), lambda i: (0, i)),
        ],
        out_specs=pl.BlockSpec((8, BLK_E), lambda i: (0, i)),
        out_shape=jax.ShapeDtypeStruct((8, E), jnp.float32),
    )(Wecat, becat, ea_t)


@functools.lru_cache(maxsize=None)
def _sc_mesh():
    return plsc.VectorSubcoreMesh(core_axis_name="c", subcore_axis_name="s",
                                  num_cores=2, num_subcores=16)


def _sc_edge_pass(nch, ch_rows, e_row0, kqv_hbm, e_hbm, ei_hbm, out_hbm,
                  src_v, dst_v, ktab, qtab, vtab, e_v, agg):
    wid = lax.axis_index("s") * 2 + lax.axis_index("c")
    base = wid * EW
    pltpu.sync_copy(ei_hbm.at[pl.ds(base, EW)], src_v)
    pltpu.sync_copy(ei_hbm.at[pl.ds(E + base, EW)], dst_v)
    kr, qr, vr = ch_rows
    zero = jnp.zeros((LANES,), jnp.float32)
    for c in range(nch):
        pltpu.sync_copy(kqv_hbm.at[pl.ds((kr + c) * NPAD, NPAD)], ktab)
        pltpu.sync_copy(kqv_hbm.at[pl.ds((qr + c) * NPAD, NPAD)], qtab)
        pltpu.sync_copy(kqv_hbm.at[pl.ds((vr + c) * NPAD, NPAD)], vtab)
        pltpu.sync_copy(e_hbm.at[pl.ds((e_row0 + c) * E + base, EW)], e_v)

        def zbody(i, _):
            for u in range(8):
                agg[pl.ds((i * 8 + u) * LANES, LANES)] = zero
            return 0
        lax.fori_loop(0, NPAD // (8 * LANES), zbody, 0)

        UNROLL = 5
        def ebody(i, _):
            for u in range(UNROLL):
                off = (i * UNROLL + u) * LANES
                sidx = src_v[pl.ds(off, LANES)]
                didx = dst_v[pl.ds(off, LANES)]
                kd = plsc.load_gather(ktab, [didx])
                qs = plsc.load_gather(qtab, [sidx])
                vs = plsc.load_gather(vtab, [sidx])
                ev = e_v[pl.ds(off, LANES)]
                z = kd + qs + ev
                g = 1.0 / (1.0 + jnp.exp(-z))
                plsc.addupdate_scatter(agg, [didx], g * vs)
            return 0
        lax.fori_loop(0, EW // (UNROLL * LANES), ebody, 0)

        pltpu.sync_copy(agg, out_hbm.at[pl.ds((c * NW + wid) * NPAD, NPAD)])


def _sc_scratch():
    return [
        pltpu.VMEM((EW,), jnp.int32),
        pltpu.VMEM((EW,), jnp.int32),
        pltpu.VMEM((NPAD,), jnp.float32),
        pltpu.VMEM((NPAD,), jnp.float32),
        pltpu.VMEM((NPAD,), jnp.float32),
        pltpu.VMEM((EW,), jnp.float32),
        pltpu.VMEM((NPAD,), jnp.float32),
    ]


@functools.lru_cache(maxsize=None)
def _sc_conv1():
    @functools.partial(
        pl.kernel,
        out_type=jax.ShapeDtypeStruct((H1 * NW * NPAD,), jnp.float32),
        mesh=_sc_mesh(),
        scratch_types=_sc_scratch(),
        compiler_params=pltpu.CompilerParams(needs_layout_passes=False),
    )
    def body(kqv_hbm, e_hbm, ei_hbm, out_hbm, *scratch):
        _sc_edge_pass(H1, (0, 5, 10), 0, kqv_hbm, e_hbm, ei_hbm, out_hbm,
                      *scratch)
    return body


@functools.lru_cache(maxsize=None)
def _sc_conv2():
    @functools.partial(
        pl.kernel,
        out_type=jax.ShapeDtypeStruct((NW * NPAD,), jnp.float32),
        mesh=_sc_mesh(),
        scratch_types=_sc_scratch(),
        compiler_params=pltpu.CompilerParams(needs_layout_passes=False),
    )
    def body(kqv_hbm, e_hbm, ei_hbm, out_hbm, *scratch):
        _sc_edge_pass(1, (0, 1, 2), 5, kqv_hbm, e_hbm, ei_hbm, out_hbm,
                      *scratch)
    return body


def _tc_norm(parts_ref, kqv_ref, batch_ref, gw_ref, gb_ref, gms_ref,
             w2_ref, b2_ref, out_ref):
    rows = parts_ref[...]
    hs = []
    for c in range(H1):
        agg_c = jnp.sum(rows[c * NW:(c + 1) * NW, :], axis=0, keepdims=True)
        hs.append(agg_c + kqv_ref[15 + c:16 + c, :])
    h = jnp.concatenate(hs, axis=0)

    bcol = batch_ref[...]
    seg = lax.broadcasted_iota(jnp.int32, (NPAD, G), 1)
    m = jnp.where(bcol == seg, 1.0, 0.0)
    cnt = jnp.maximum(jnp.sum(m, axis=0, keepdims=True), 1.0)
    sums = lax.dot_general(h, m, (((1,), (0,)), ((), ())),
                           preferred_element_type=jnp.float32, precision=HIGHEST)
    mean = sums / cnt
    mean_x = lax.dot_general(mean, m, (((1,), (1,)), ((), ())),
                             preferred_element_type=jnp.float32, precision=HIGHEST)
    cent = h - gms_ref[...] * mean_x
    var = lax.dot_general(cent * cent, m, (((1,), (0,)), ((), ())),
                          preferred_element_type=jnp.float32, precision=HIGHEST) / cnt
    std = jnp.sqrt(var + 1e-5)
    std_x = lax.dot_general(std, m, (((1,), (1,)), ((), ())),
                            preferred_element_type=jnp.float32, precision=HIGHEST)
    normed = gw_ref[...] * cent / std_x + gb_ref[...]
    h2 = jnp.maximum(normed, 0.0)
    rows4 = lax.dot_general(w2_ref[...], h2, (((0,), (0,)), ((), ())),
                            preferred_element_type=jnp.float32, precision=HIGHEST)
    out_ref[...] = rows4 + b2_ref[...]


def _norm_stage(parts1_2d, kqv_cm, batch2d, gw, gb, gms, W2cat, b2cat):
    return pl.pallas_call(
        _tc_norm,
        out_shape=jax.ShapeDtypeStruct((4, NPAD), jnp.float32),
    )(parts1_2d, kqv_cm, batch2d, gw, gb, gms, W2cat, b2cat)


def _tc_final(parts_ref, kqvs_ref, out_ref):
    s = jnp.sum(parts_ref[...], axis=0, keepdims=True)
    z = s + kqvs_ref[3:4, :]
    out_ref[...] = 1.0 / (1.0 + jnp.exp(-z))


def _final_stage(parts2, kqvs2):
    return pl.pallas_call(
        _tc_final,
        out_shape=jax.ShapeDtypeStruct((1, NPAD), jnp.float32),
    )(parts2, kqvs2)


def kernel(x, edge_index, edge_attr, batch_idx, Wk1, bk1, Wq1, bq1, Wv1, bv1,
           We1, be1, Ws1, b1, gw, gb, gms, Wk2, bk2, Wq2, bq2, Wv2, bv2,
           We2, be2, Ws2, b2):
    x_pad = jnp.pad(x, ((0, NPAD - N), (0, 0)))
    Wcat = jnp.concatenate(
        [Wk1, Wq1, Wv1, Ws1, jnp.zeros((D, 4), jnp.float32)], axis=1)
    bcat = jnp.concatenate(
        [bk1, bq1, bv1, b1, jnp.zeros((4,), jnp.float32)])[:, None]
    kqv_cm = _node_proj(x_pad, Wcat, bcat)

    Wecat = jnp.concatenate(
        [We1, We2, jnp.zeros((ED, 2), jnp.float32)], axis=1)
    becat = jnp.concatenate(
        [be1, be2, jnp.zeros((2,), jnp.float32)])[:, None]
    e_cm = _edge_proj(Wecat, becat, edge_attr.T)

    kqv_flat = kqv_cm.reshape(-1)
    e_flat = e_cm.reshape(-1)
    ei_flat = edge_index.reshape(-1)
    parts1 = _sc_conv1()(kqv_flat, e_flat, ei_flat)

    batch2d = jnp.pad(batch_idx, (0, NPAD - N), constant_values=G)[:, None]
    W2cat = jnp.concatenate([Wk2, Wq2, Wv2, Ws2], axis=1)
    b2cat = jnp.concatenate([bk2, bq2, bv2, b2])[:, None]
    kqvs2 = _norm_stage(parts1.reshape(H1 * NW, NPAD), kqv_cm, batch2d,
                        gw[:, None], gb[:, None], gms[:, None], W2cat, b2cat)

    parts2 = _sc_conv2()(kqvs2.reshape(-1), e_flat, ei_flat)
    out = _final_stage(parts2.reshape(NW, NPAD), kqvs2)
    return out[0, :N].reshape(N, 1)


# speedup vs baseline: 60.3528x; 1.8620x over previous
import functools

import jax
import jax.numpy as jnp
from jax import lax
from jax.experimental import pallas as pl
from jax.experimental.pallas import tpu as pltpu
from jax.experimental.pallas import tpu_sc as plsc

N = 10000
E = 320000
D = 128
ED = 16
H1 = 5
H2 = 1
G = 64

NPAD = 10240
NW = 32
EW = E // NW
LANES = 16
HIGHEST = lax.Precision.HIGHEST


def _tc_node_proj(w_ref, b_ref, x_ref, out_ref):
    out = lax.dot_general(w_ref[...], x_ref[...], (((0,), (1,)), ((), ())),
                          preferred_element_type=jnp.float32, precision=HIGHEST)
    out_ref[...] = out + b_ref[...]


def _node_proj(x_pad, Wcat, bcat):
    return pl.pallas_call(
        _tc_node_proj,
        out_shape=jax.ShapeDtypeStruct((24, NPAD), jnp.float32),
    )(Wcat, bcat, x_pad)


BLK_E = 6400


def _tc_edge_proj(w_ref, b_ref, ea_ref, out_ref):
    out = lax.dot_general(w_ref[...], ea_ref[...], (((0,), (0,)), ((), ())),
                          preferred_element_type=jnp.float32, precision=HIGHEST)
    out_ref[...] = out + b_ref[...]


def _edge_proj(Wecat, becat, ea_t):
    grid = (E // BLK_E,)
    return pl.pallas_call(
        _tc_edge_proj,
        grid=grid,
        in_specs=[
            pl.BlockSpec((ED, 8), lambda i: (0, 0)),
            pl.BlockSpec((8, 1), lambda i: (0, 0)),
            pl.BlockSpec((ED, BLK_E), lambda i: (0, i)),
        ],
        out_specs=pl.BlockSpec((8, BLK_E), lambda i: (0, i)),
        out_shape=jax.ShapeDtypeStruct((8, E), jnp.float32),
    )(Wecat, becat, ea_t)


@functools.lru_cache(maxsize=None)
def _sc_mesh():
    return plsc.VectorSubcoreMesh(core_axis_name="c", subcore_axis_name="s",
                                  num_cores=2, num_subcores=16)


def _sc_edge_pass(nch, ch_rows, e_row0, kqv_hbm, e_hbm, ei_hbm, out_hbm,
                  src_v, dst_v, ktab0, qtab0, vtab0, e_v0,
                  ktab1, qtab1, vtab1, e_v1, agg,
                  sem_idx, sem_a, sem_b):
    wid = lax.axis_index("s") * 2 + lax.axis_index("c")
    base = wid * EW
    cp_s = pltpu.async_copy(ei_hbm.at[pl.ds(base, EW)], src_v, sem_idx)
    cp_d = pltpu.async_copy(ei_hbm.at[pl.ds(E + base, EW)], dst_v, sem_idx)
    kr, qr, vr = ch_rows
    sems = (sem_a, sem_b)
    bufs = ((ktab0, qtab0, vtab0, e_v0), (ktab1, qtab1, vtab1, e_v1))

    def start_tables(c, b):
        kt, qt, vt, ev = bufs[b]
        return [
            pltpu.async_copy(kqv_hbm.at[pl.ds((kr + c) * NPAD, NPAD)],
                             kt, sems[b]),
            pltpu.async_copy(kqv_hbm.at[pl.ds((qr + c) * NPAD, NPAD)],
                             qt, sems[b]),
            pltpu.async_copy(kqv_hbm.at[pl.ds((vr + c) * NPAD, NPAD)],
                             vt, sems[b]),
            pltpu.async_copy(e_hbm.at[pl.ds((e_row0 + c) * E + base, EW)],
                             ev, sems[b]),
        ]

    pending = start_tables(0, 0)
    cp_s.wait()
    cp_d.wait()
    zero = jnp.zeros((LANES,), jnp.float32)
    for c in range(nch):
        b = c & 1
        for cp in pending:
            cp.wait()
        if c + 1 < nch:
            pending = start_tables(c + 1, 1 - b)

        def zbody(i, _):
            for u in range(8):
                agg[pl.ds((i * 8 + u) * LANES, LANES)] = zero
            return 0
        lax.fori_loop(0, NPAD // (8 * LANES), zbody, 0)

        kt, qt, vt, evb = bufs[b]

        @plsc.parallel_loop(0, EW // LANES, 1, unroll=5)
        def _(i):
            off = i * LANES
            sidx = src_v[pl.ds(off, LANES)]
            didx = dst_v[pl.ds(off, LANES)]
            kd = plsc.load_gather(kt, [didx])
            qs = plsc.load_gather(qt, [sidx])
            vs = plsc.load_gather(vt, [sidx])
            ev = evb[pl.ds(off, LANES)]
            z = kd + qs + ev
            g = 1.0 / (1.0 + jnp.exp(-z))
            plsc.addupdate_scatter(agg, [didx], g * vs)

        pltpu.sync_copy(agg, out_hbm.at[pl.ds((c * NW + wid) * NPAD, NPAD)])


def _sc_scratch():
    return [
        pltpu.VMEM((EW,), jnp.int32),
        pltpu.VMEM((EW,), jnp.int32),
        pltpu.VMEM((NPAD,), jnp.float32),
        pltpu.VMEM((NPAD,), jnp.float32),
        pltpu.VMEM((NPAD,), jnp.float32),
        pltpu.VMEM((EW,), jnp.float32),
        pltpu.VMEM((NPAD,), jnp.float32),
        pltpu.VMEM((NPAD,), jnp.float32),
        pltpu.VMEM((NPAD,), jnp.float32),
        pltpu.VMEM((EW,), jnp.float32),
        pltpu.VMEM((NPAD,), jnp.float32),
        pltpu.SemaphoreType.DMA,
        pltpu.SemaphoreType.DMA,
        pltpu.SemaphoreType.DMA,
    ]


@functools.lru_cache(maxsize=None)
def _sc_conv1():
    @functools.partial(
        pl.kernel,
        out_type=jax.ShapeDtypeStruct((H1 * NW * NPAD,), jnp.float32),
        mesh=_sc_mesh(),
        scratch_types=_sc_scratch(),
        compiler_params=pltpu.CompilerParams(needs_layout_passes=False),
    )
    def body(kqv_hbm, e_hbm, ei_hbm, out_hbm, *scratch):
        _sc_edge_pass(H1, (0, 5, 10), 0, kqv_hbm, e_hbm, ei_hbm, out_hbm,
                      *scratch)
    return body


@functools.lru_cache(maxsize=None)
def _sc_conv2():
    @functools.partial(
        pl.kernel,
        out_type=jax.ShapeDtypeStruct((NW * NPAD,), jnp.float32),
        mesh=_sc_mesh(),
        scratch_types=_sc_scratch(),
        compiler_params=pltpu.CompilerParams(needs_layout_passes=False),
    )
    def body(kqv_hbm, e_hbm, ei_hbm, out_hbm, *scratch):
        _sc_edge_pass(1, (0, 1, 2), 5, kqv_hbm, e_hbm, ei_hbm, out_hbm,
                      *scratch)
    return body


def _tc_norm(parts_ref, kqv_ref, batch_ref, gw_ref, gb_ref, gms_ref,
             w2_ref, b2_ref, out_ref):
    rows = parts_ref[...]
    hs = []
    for c in range(H1):
        agg_c = jnp.sum(rows[c * NW:(c + 1) * NW, :], axis=0, keepdims=True)
        hs.append(agg_c + kqv_ref[15 + c:16 + c, :])
    h = jnp.concatenate(hs, axis=0)

    bcol = batch_ref[...]
    seg = lax.broadcasted_iota(jnp.int32, (NPAD, G), 1)
    m = jnp.where(bcol == seg, 1.0, 0.0)
    cnt = jnp.maximum(jnp.sum(m, axis=0, keepdims=True), 1.0)
    sums = lax.dot_general(h, m, (((1,), (0,)), ((), ())),
                           preferred_element_type=jnp.float32, precision=HIGHEST)
    mean = sums / cnt
    mean_x = lax.dot_general(mean, m, (((1,), (1,)), ((), ())),
                             preferred_element_type=jnp.float32, precision=HIGHEST)
    cent = h - gms_ref[...] * mean_x
    var = lax.dot_general(cent * cent, m, (((1,), (0,)), ((), ())),
                          preferred_element_type=jnp.float32, precision=HIGHEST) / cnt
    std = jnp.sqrt(var + 1e-5)
    std_x = lax.dot_general(std, m, (((1,), (1,)), ((), ())),
                            preferred_element_type=jnp.float32, precision=HIGHEST)
    normed = gw_ref[...] * cent / std_x + gb_ref[...]
    h2 = jnp.maximum(normed, 0.0)
    rows4 = lax.dot_general(w2_ref[...], h2, (((0,), (0,)), ((), ())),
                            preferred_element_type=jnp.float32, precision=HIGHEST)
    out_ref[...] = rows4 + b2_ref[...]


def _norm_stage(parts1_2d, kqv_cm, batch2d, gw, gb, gms, W2cat, b2cat):
    return pl.pallas_call(
        _tc_norm,
        out_shape=jax.ShapeDtypeStruct((4, NPAD), jnp.float32),
    )(parts1_2d, kqv_cm, batch2d, gw, gb, gms, W2cat, b2cat)


def _tc_final(parts_ref, kqvs_ref, out_ref):
    s = jnp.sum(parts_ref[...], axis=0, keepdims=True)
    z = s + kqvs_ref[3:4, :]
    out_ref[...] = 1.0 / (1.0 + jnp.exp(-z))


def _final_stage(parts2, kqvs2):
    return pl.pallas_call(
        _tc_final,
        out_shape=jax.ShapeDtypeStruct((1, NPAD), jnp.float32),
    )(parts2, kqvs2)


def kernel(x, edge_index, edge_attr, batch_idx, Wk1, bk1, Wq1, bq1, Wv1, bv1,
           We1, be1, Ws1, b1, gw, gb, gms, Wk2, bk2, Wq2, bq2, Wv2, bv2,
           We2, be2, Ws2, b2):
    x_pad = jnp.pad(x, ((0, NPAD - N), (0, 0)))
    Wcat = jnp.concatenate(
        [Wk1, Wq1, Wv1, Ws1, jnp.zeros((D, 4), jnp.float32)], axis=1)
    bcat = jnp.concatenate(
        [bk1, bq1, bv1, b1, jnp.zeros((4,), jnp.float32)])[:, None]
    kqv_cm = _node_proj(x_pad, Wcat, bcat)

    Wecat = jnp.concatenate(
        [We1, We2, jnp.zeros((ED, 2), jnp.float32)], axis=1)
    becat = jnp.concatenate(
        [be1, be2, jnp.zeros((2,), jnp.float32)])[:, None]
    e_cm = _edge_proj(Wecat, becat, edge_attr.T)

    kqv_flat = kqv_cm.reshape(-1)
    e_flat = e_cm.reshape(-1)
    ei_flat = edge_index.reshape(-1)
    parts1 = _sc_conv1()(kqv_flat, e_flat, ei_flat)

    batch2d = jnp.pad(batch_idx, (0, NPAD - N), constant_values=G)[:, None]
    W2cat = jnp.concatenate([Wk2, Wq2, Wv2, Ws2], axis=1)
    b2cat = jnp.concatenate([bk2, bq2, bv2, b2])[:, None]
    kqvs2 = _norm_stage(parts1.reshape(H1 * NW, NPAD), kqv_cm, batch2d,
                        gw[:, None], gb[:, None], gms[:, None], W2cat, b2cat)

    parts2 = _sc_conv2()(kqvs2.reshape(-1), e_flat, ei_flat)
    out = _final_stage(parts2.reshape(NW, NPAD), kqvs2)
    return out[0, :N].reshape(N, 1)


# speedup vs baseline: 62.2707x; 1.0318x over previous
import functools

import jax
import jax.numpy as jnp
from jax import lax
from jax.experimental import pallas as pl
from jax.experimental.pallas import tpu as pltpu
from jax.experimental.pallas import tpu_sc as plsc

N = 10000
E = 320000
D = 128
ED = 16
H1 = 5
H2 = 1
G = 64

NPAD = 10240
NW = 32
EW = E // NW
LANES = 16
HIGHEST = lax.Precision.HIGHEST


def _tc_node_proj(w_ref, b_ref, x_ref, out_ref):
    out = lax.dot_general(w_ref[...], x_ref[...], (((0,), (1,)), ((), ())),
                          preferred_element_type=jnp.float32, precision=HIGHEST)
    out_ref[...] = out + b_ref[...]


def _node_proj(x_pad, Wcat, bcat):
    return pl.pallas_call(
        _tc_node_proj,
        out_shape=jax.ShapeDtypeStruct((24, NPAD), jnp.float32),
    )(Wcat, bcat, x_pad)


BLK_E = 6400


def _tc_edge_proj(w_ref, b_ref, ea_ref, out_ref):
    out = lax.dot_general(w_ref[...], ea_ref[...], (((0,), (0,)), ((), ())),
                          preferred_element_type=jnp.float32, precision=HIGHEST)
    out_ref[...] = out + b_ref[...]


def _edge_proj(Wecat, becat, ea_t):
    grid = (E // BLK_E,)
    return pl.pallas_call(
        _tc_edge_proj,
        grid=grid,
        in_specs=[
            pl.BlockSpec((ED, 8), lambda i: (0, 0)),
            pl.BlockSpec((8, 1), lambda i: (0, 0)),
            pl.BlockSpec((ED, BLK_E), lambda i: (0, i)),
        ],
        out_specs=pl.BlockSpec((8, BLK_E), lambda i: (0, i)),
        out_shape=jax.ShapeDtypeStruct((8, E), jnp.float32),
    )(Wecat, becat, ea_t)


@functools.lru_cache(maxsize=None)
def _sc_mesh():
    return plsc.VectorSubcoreMesh(core_axis_name="c", subcore_axis_name="s",
                                  num_cores=2, num_subcores=16)


def _sc_edge_pass(nch, ch_rows, e_row0, kqv_hbm, e_hbm, ei_hbm, out_hbm,
                  src_v, dst_v, ktab0, qtab0, vtab0, e_v0,
                  ktab1, qtab1, vtab1, e_v1, agg,
                  sem_idx, sem_a, sem_b):
    wid = lax.axis_index("s") * 2 + lax.axis_index("c")
    base = wid * EW
    cp_s = pltpu.async_copy(ei_hbm.at[pl.ds(base, EW)], src_v, sem_idx)
    cp_d = pltpu.async_copy(ei_hbm.at[pl.ds(E + base, EW)], dst_v, sem_idx)
    kr, qr, vr = ch_rows
    sems = (sem_a, sem_b)
    bufs = ((ktab0, qtab0, vtab0, e_v0), (ktab1, qtab1, vtab1, e_v1))

    def start_tables(c, b):
        kt, qt, vt, ev = bufs[b]
        return [
            pltpu.async_copy(kqv_hbm.at[pl.ds((kr + c) * NPAD, NPAD)],
                             kt, sems[b]),
            pltpu.async_copy(kqv_hbm.at[pl.ds((qr + c) * NPAD, NPAD)],
                             qt, sems[b]),
            pltpu.async_copy(kqv_hbm.at[pl.ds((vr + c) * NPAD, NPAD)],
                             vt, sems[b]),
            pltpu.async_copy(e_hbm.at[pl.ds((e_row0 + c) * E + base, EW)],
                             ev, sems[b]),
        ]

    pending = start_tables(0, 0)
    cp_s.wait()
    cp_d.wait()
    zero = jnp.zeros((LANES,), jnp.float32)
    for c in range(nch):
        b = c & 1
        for cp in pending:
            cp.wait()
        if c + 1 < nch:
            pending = start_tables(c + 1, 1 - b)

        def zbody(i, _):
            for u in range(8):
                agg[pl.ds((i * 8 + u) * LANES, LANES)] = zero
            return 0
        lax.fori_loop(0, NPAD // (8 * LANES), zbody, 0)

        kt, qt, vt, evb = bufs[b]

        @plsc.parallel_loop(0, EW // LANES, 1, unroll=10)
        def _(i):
            off = i * LANES
            sidx = src_v[pl.ds(off, LANES)]
            didx = dst_v[pl.ds(off, LANES)]
            kd = plsc.load_gather(kt, [didx])
            qs = plsc.load_gather(qt, [sidx])
            vs = plsc.load_gather(vt, [sidx])
            ev = evb[pl.ds(off, LANES)]
            z = kd + qs + ev
            g = 1.0 / (1.0 + jnp.exp(-z))
            plsc.addupdate_scatter(agg, [didx], g * vs)

        pltpu.sync_copy(agg, out_hbm.at[pl.ds((c * NW + wid) * NPAD, NPAD)])


def _sc_scratch():
    return [
        pltpu.VMEM((EW,), jnp.int32),
        pltpu.VMEM((EW,), jnp.int32),
        pltpu.VMEM((NPAD,), jnp.float32),
        pltpu.VMEM((NPAD,), jnp.float32),
        pltpu.VMEM((NPAD,), jnp.float32),
        pltpu.VMEM((EW,), jnp.float32),
        pltpu.VMEM((NPAD,), jnp.float32),
        pltpu.VMEM((NPAD,), jnp.float32),
        pltpu.VMEM((NPAD,), jnp.float32),
        pltpu.VMEM((EW,), jnp.float32),
        pltpu.VMEM((NPAD,), jnp.float32),
        pltpu.SemaphoreType.DMA,
        pltpu.SemaphoreType.DMA,
        pltpu.SemaphoreType.DMA,
    ]


@functools.lru_cache(maxsize=None)
def _sc_conv1():
    @functools.partial(
        pl.kernel,
        out_type=jax.ShapeDtypeStruct((H1 * NW * NPAD,), jnp.float32),
        mesh=_sc_mesh(),
        scratch_types=_sc_scratch(),
        compiler_params=pltpu.CompilerParams(needs_layout_passes=False),
    )
    def body(kqv_hbm, e_hbm, ei_hbm, out_hbm, *scratch):
        _sc_edge_pass(H1, (0, 5, 10), 0, kqv_hbm, e_hbm, ei_hbm, out_hbm,
                      *scratch)
    return body


@functools.lru_cache(maxsize=None)
def _sc_conv2():
    @functools.partial(
        pl.kernel,
        out_type=jax.ShapeDtypeStruct((NW * NPAD,), jnp.float32),
        mesh=_sc_mesh(),
        scratch_types=_sc_scratch(),
        compiler_params=pltpu.CompilerParams(needs_layout_passes=False),
    )
    def body(kqv_hbm, e_hbm, ei_hbm, out_hbm, *scratch):
        _sc_edge_pass(1, (0, 1, 2), 5, kqv_hbm, e_hbm, ei_hbm, out_hbm,
                      *scratch)
    return body


def _tc_norm(parts_ref, kqv_ref, batch_ref, gw_ref, gb_ref, gms_ref,
             w2_ref, b2_ref, out_ref):
    rows = parts_ref[...]
    hs = []
    for c in range(H1):
        agg_c = jnp.sum(rows[c * NW:(c + 1) * NW, :], axis=0, keepdims=True)
        hs.append(agg_c + kqv_ref[15 + c:16 + c, :])
    h = jnp.concatenate(hs, axis=0)

    bcol = batch_ref[...]
    seg = lax.broadcasted_iota(jnp.int32, (NPAD, G), 1)
    m = jnp.where(bcol == seg, 1.0, 0.0)
    stack11 = jnp.concatenate([h, h * h, jnp.ones((1, NPAD), jnp.float32)],
                              axis=0)
    s11 = lax.dot_general(stack11, m, (((1,), (0,)), ((), ())),
                          preferred_element_type=jnp.float32, precision=HIGHEST)
    cnt = jnp.maximum(s11[2 * H1:2 * H1 + 1], 1.0)
    mean = s11[0:H1] / cnt
    msq = s11[H1:2 * H1] / cnt
    gms = gms_ref[...]
    var = msq - (2.0 * gms - gms * gms) * mean * mean
    std = jnp.sqrt(var + 1e-5)
    a = gw_ref[...] / std
    bco = gb_ref[...] - gw_ref[...] * gms * mean / std
    ab = jnp.concatenate([a, bco], axis=0)
    ab_x = lax.dot_general(ab, m, (((1,), (1,)), ((), ())),
                           preferred_element_type=jnp.float32, precision=HIGHEST)
    normed = ab_x[0:H1] * h + ab_x[H1:2 * H1]
    h2 = jnp.maximum(normed, 0.0)
    rows4 = lax.dot_general(w2_ref[...], h2, (((0,), (0,)), ((), ())),
                            preferred_element_type=jnp.float32, precision=HIGHEST)
    out_ref[...] = rows4 + b2_ref[...]


def _norm_stage(parts1_2d, kqv_cm, batch2d, gw, gb, gms, W2cat, b2cat):
    return pl.pallas_call(
        _tc_norm,
        out_shape=jax.ShapeDtypeStruct((4, NPAD), jnp.float32),
    )(parts1_2d, kqv_cm, batch2d, gw, gb, gms, W2cat, b2cat)


def _tc_final(parts_ref, kqvs_ref, out_ref):
    s = jnp.sum(parts_ref[...], axis=0, keepdims=True)
    z = s + kqvs_ref[3:4, :]
    out_ref[...] = 1.0 / (1.0 + jnp.exp(-z))


def _final_stage(parts2, kqvs2):
    return pl.pallas_call(
        _tc_final,
        out_shape=jax.ShapeDtypeStruct((1, NPAD), jnp.float32),
    )(parts2, kqvs2)


def kernel(x, edge_index, edge_attr, batch_idx, Wk1, bk1, Wq1, bq1, Wv1, bv1,
           We1, be1, Ws1, b1, gw, gb, gms, Wk2, bk2, Wq2, bq2, Wv2, bv2,
           We2, be2, Ws2, b2):
    x_pad = jnp.pad(x, ((0, NPAD - N), (0, 0)))
    Wcat = jnp.concatenate(
        [Wk1, Wq1, Wv1, Ws1, jnp.zeros((D, 4), jnp.float32)], axis=1)
    bcat = jnp.concatenate(
        [bk1, bq1, bv1, b1, jnp.zeros((4,), jnp.float32)])[:, None]
    kqv_cm = _node_proj(x_pad, Wcat, bcat)

    Wecat = jnp.concatenate(
        [We1, We2, jnp.zeros((ED, 2), jnp.float32)], axis=1)
    becat = jnp.concatenate(
        [be1, be2, jnp.zeros((2,), jnp.float32)])[:, None]
    e_cm = _edge_proj(Wecat, becat, edge_attr.T)

    kqv_flat = kqv_cm.reshape(-1)
    e_flat = e_cm.reshape(-1)
    ei_flat = edge_index.reshape(-1)
    parts1 = _sc_conv1()(kqv_flat, e_flat, ei_flat)

    batch2d = jnp.pad(batch_idx, (0, NPAD - N), constant_values=G)[:, None]
    W2cat = jnp.concatenate([Wk2, Wq2, Wv2, Ws2], axis=1)
    b2cat = jnp.concatenate([bk2, bq2, bv2, b2])[:, None]
    kqvs2 = _norm_stage(parts1.reshape(H1 * NW, NPAD), kqv_cm, batch2d,
                        gw[:, None], gb[:, None], gms[:, None], W2cat, b2cat)

    parts2 = _sc_conv2()(kqvs2.reshape(-1), e_flat, ei_flat)
    out = _final_stage(parts2.reshape(NW, NPAD), kqvs2)
    return out[0, :N].reshape(N, 1)


# speedup vs baseline: 64.7546x; 1.0399x over previous
import functools

import jax
import jax.numpy as jnp
from jax import lax
from jax.experimental import pallas as pl
from jax.experimental.pallas import tpu as pltpu
from jax.experimental.pallas import tpu_sc as plsc

N = 10000
E = 320000
D = 128
ED = 16
H1 = 5
H2 = 1
G = 64

NPAD = 10240
NW = 32
EW = E // NW
LANES = 16
HIGHEST = lax.Precision.HIGHEST


def _tc_node_proj(w_ref, b_ref, x_ref, out_ref):
    out = lax.dot_general(w_ref[...], x_ref[...], (((0,), (1,)), ((), ())),
                          preferred_element_type=jnp.float32, precision=HIGHEST)
    out_ref[...] = out + b_ref[...]


def _node_proj(x_pad, Wcat, bcat):
    return pl.pallas_call(
        _tc_node_proj,
        out_shape=jax.ShapeDtypeStruct((24, NPAD), jnp.float32),
    )(Wcat, bcat, x_pad)


BLK_E = 6400


def _tc_edge_proj(w_ref, b_ref, ea_ref, *out_refs):
    out = lax.dot_general(w_ref[...], ea_ref[...], (((0,), (0,)), ((), ())),
                          preferred_element_type=jnp.float32, precision=HIGHEST)
    out = out + b_ref[...]
    i = pl.program_id(0)
    for c, ref in enumerate(out_refs):
        ref[pl.ds(i * BLK_E, BLK_E)] = out[c]


def _edge_proj(Wecat, becat, ea_t):
    grid = (E // BLK_E,)
    return pl.pallas_call(
        _tc_edge_proj,
        grid=grid,
        in_specs=[
            pl.BlockSpec((ED, 6), lambda i: (0, 0)),
            pl.BlockSpec((6, 1), lambda i: (0, 0)),
            pl.BlockSpec((ED, BLK_E), lambda i: (0, i)),
        ],
        out_specs=[pl.BlockSpec((E,), lambda i: (0,))] * 6,
        out_shape=[jax.ShapeDtypeStruct((E,), jnp.float32)] * 6,
    )(Wecat, becat, ea_t)


@functools.lru_cache(maxsize=None)
def _sc_mesh():
    return plsc.VectorSubcoreMesh(core_axis_name="c", subcore_axis_name="s",
                                  num_cores=2, num_subcores=16)


def _sc_edge_pass(nch, ch_rows, e_list, kqv_hbm, ei_hbm, out_hbm,
                  src_v, dst_v, ktab0, qtab0, vtab0, e_v0,
                  ktab1, qtab1, vtab1, e_v1, agg,
                  sem_idx, sem_a, sem_b):
    wid = lax.axis_index("s") * 2 + lax.axis_index("c")
    base = wid * EW
    cp_s = pltpu.async_copy(ei_hbm.at[pl.ds(base, EW)], src_v, sem_idx)
    cp_d = pltpu.async_copy(ei_hbm.at[pl.ds(E + base, EW)], dst_v, sem_idx)
    kr, qr, vr = ch_rows
    sems = (sem_a, sem_b)
    bufs = ((ktab0, qtab0, vtab0, e_v0), (ktab1, qtab1, vtab1, e_v1))

    def start_tables(c, b):
        kt, qt, vt, ev = bufs[b]
        return [
            pltpu.async_copy(kqv_hbm.at[pl.ds((kr + c) * NPAD, NPAD)],
                             kt, sems[b]),
            pltpu.async_copy(kqv_hbm.at[pl.ds((qr + c) * NPAD, NPAD)],
                             qt, sems[b]),
            pltpu.async_copy(kqv_hbm.at[pl.ds((vr + c) * NPAD, NPAD)],
                             vt, sems[b]),
            pltpu.async_copy(e_list[c].at[pl.ds(base, EW)], ev, sems[b]),
        ]

    pending = start_tables(0, 0)
    cp_s.wait()
    cp_d.wait()
    zero = jnp.zeros((LANES,), jnp.float32)
    for c in range(nch):
        b = c & 1
        for cp in pending:
            cp.wait()
        if c + 1 < nch:
            pending = start_tables(c + 1, 1 - b)

        def zbody(i, _):
            for u in range(8):
                agg[pl.ds((i * 8 + u) * LANES, LANES)] = zero
            return 0
        lax.fori_loop(0, NPAD // (8 * LANES), zbody, 0)

        kt, qt, vt, evb = bufs[b]

        @plsc.parallel_loop(0, EW // LANES, 1, unroll=10)
        def _(i):
            off = i * LANES
            sidx = src_v[pl.ds(off, LANES)]
            didx = dst_v[pl.ds(off, LANES)]
            kd = plsc.load_gather(kt, [didx])
            qs = plsc.load_gather(qt, [sidx])
            vs = plsc.load_gather(vt, [sidx])
            ev = evb[pl.ds(off, LANES)]
            z = kd + qs + ev
            g = 1.0 / (1.0 + jnp.exp(-z))
            plsc.addupdate_scatter(agg, [didx], g * vs)

        pltpu.sync_copy(agg, out_hbm.at[pl.ds((c * NW + wid) * NPAD, NPAD)])


def _sc_scratch():
    return [
        pltpu.VMEM((EW,), jnp.int32),
        pltpu.VMEM((EW,), jnp.int32),
        pltpu.VMEM((NPAD,), jnp.float32),
        pltpu.VMEM((NPAD,), jnp.float32),
        pltpu.VMEM((NPAD,), jnp.float32),
        pltpu.VMEM((EW,), jnp.float32),
        pltpu.VMEM((NPAD,), jnp.float32),
        pltpu.VMEM((NPAD,), jnp.float32),
        pltpu.VMEM((NPAD,), jnp.float32),
        pltpu.VMEM((EW,), jnp.float32),
        pltpu.VMEM((NPAD,), jnp.float32),
        pltpu.SemaphoreType.DMA,
        pltpu.SemaphoreType.DMA,
        pltpu.SemaphoreType.DMA,
    ]


@functools.lru_cache(maxsize=None)
def _sc_conv1():
    @functools.partial(
        pl.kernel,
        out_type=jax.ShapeDtypeStruct((H1 * NW * NPAD,), jnp.float32),
        mesh=_sc_mesh(),
        scratch_types=_sc_scratch(),
        compiler_params=pltpu.CompilerParams(needs_layout_passes=False),
    )
    def body(kqv_hbm, e0, e1, e2, e3, e4, ei_hbm, out_hbm, *scratch):
        _sc_edge_pass(H1, (0, 5, 10), (e0, e1, e2, e3, e4), kqv_hbm,
                      ei_hbm, out_hbm, *scratch)
    return body


@functools.lru_cache(maxsize=None)
def _sc_conv2():
    @functools.partial(
        pl.kernel,
        out_type=jax.ShapeDtypeStruct((NW * NPAD,), jnp.float32),
        mesh=_sc_mesh(),
        scratch_types=_sc_scratch(),
        compiler_params=pltpu.CompilerParams(needs_layout_passes=False),
    )
    def body(kqv_hbm, e5, ei_hbm, out_hbm, *scratch):
        _sc_edge_pass(1, (0, 1, 2), (e5,), kqv_hbm, ei_hbm, out_hbm,
                      *scratch)
    return body


def _tc_norm(parts_ref, kqv_ref, batch_ref, gw_ref, gb_ref, gms_ref,
             w2_ref, b2_ref, out_ref):
    rows = parts_ref[...]
    hs = []
    for c in range(H1):
        agg_c = jnp.sum(rows[c * NW:(c + 1) * NW, :], axis=0, keepdims=True)
        hs.append(agg_c + kqv_ref[15 + c:16 + c, :])
    h = jnp.concatenate(hs, axis=0)

    bcol = batch_ref[...]
    seg = lax.broadcasted_iota(jnp.int32, (NPAD, G), 1)
    m = jnp.where(bcol == seg, 1.0, 0.0)
    stack11 = jnp.concatenate([h, h * h, jnp.ones((1, NPAD), jnp.float32)],
                              axis=0)
    s11 = lax.dot_general(stack11, m, (((1,), (0,)), ((), ())),
                          preferred_element_type=jnp.float32, precision=HIGHEST)
    cnt = jnp.maximum(s11[2 * H1:2 * H1 + 1], 1.0)
    mean = s11[0:H1] / cnt
    msq = s11[H1:2 * H1] / cnt
    gms = gms_ref[...]
    var = msq - (2.0 * gms - gms * gms) * mean * mean
    std = jnp.sqrt(var + 1e-5)
    a = gw_ref[...] / std
    bco = gb_ref[...] - gw_ref[...] * gms * mean / std
    ab = jnp.concatenate([a, bco], axis=0)
    ab_x = lax.dot_general(ab, m, (((1,), (1,)), ((), ())),
                           preferred_element_type=jnp.float32, precision=HIGHEST)
    normed = ab_x[0:H1] * h + ab_x[H1:2 * H1]
    h2 = jnp.maximum(normed, 0.0)
    rows4 = lax.dot_general(w2_ref[...], h2, (((0,), (0,)), ((), ())),
                            preferred_element_type=jnp.float32, precision=HIGHEST)
    out_ref[...] = rows4 + b2_ref[...]


def _norm_stage(parts1_2d, kqv_cm, batch2d, gw, gb, gms, W2cat, b2cat):
    return pl.pallas_call(
        _tc_norm,
        out_shape=jax.ShapeDtypeStruct((4, NPAD), jnp.float32),
    )(parts1_2d, kqv_cm, batch2d, gw, gb, gms, W2cat, b2cat)


def _tc_final(parts_ref, kqvs_ref, out_ref):
    s = jnp.sum(parts_ref[...], axis=0, keepdims=True)
    z = s + kqvs_ref[3:4, :]
    out_ref[...] = 1.0 / (1.0 + jnp.exp(-z))


def _final_stage(parts2, kqvs2):
    return pl.pallas_call(
        _tc_final,
        out_shape=jax.ShapeDtypeStruct((1, NPAD), jnp.float32),
    )(parts2, kqvs2)


def kernel(x, edge_index, edge_attr, batch_idx, Wk1, bk1, Wq1, bq1, Wv1, bv1,
           We1, be1, Ws1, b1, gw, gb, gms, Wk2, bk2, Wq2, bq2, Wv2, bv2,
           We2, be2, Ws2, b2):
    x_pad = jnp.pad(x, ((0, NPAD - N), (0, 0)))
    Wcat = jnp.concatenate(
        [Wk1, Wq1, Wv1, Ws1, jnp.zeros((D, 4), jnp.float32)], axis=1)
    bcat = jnp.concatenate(
        [bk1, bq1, bv1, b1, jnp.zeros((4,), jnp.float32)])[:, None]
    kqv_cm = _node_proj(x_pad, Wcat, bcat)

    Wecat = jnp.concatenate([We1, We2], axis=1)
    becat = jnp.concatenate([be1, be2])[:, None]
    e_ch = _edge_proj(Wecat, becat, edge_attr.T)

    kqv_flat = kqv_cm.reshape(-1)
    ei_flat = edge_index.reshape(-1)
    parts1 = _sc_conv1()(kqv_flat, e_ch[0], e_ch[1], e_ch[2], e_ch[3],
                         e_ch[4], ei_flat)

    batch2d = jnp.pad(batch_idx, (0, NPAD - N), constant_values=G)[:, None]
    W2cat = jnp.concatenate([Wk2, Wq2, Wv2, Ws2], axis=1)
    b2cat = jnp.concatenate([bk2, bq2, bv2, b2])[:, None]
    kqvs2 = _norm_stage(parts1.reshape(H1 * NW, NPAD), kqv_cm, batch2d,
                        gw[:, None], gb[:, None], gms[:, None], W2cat, b2cat)

    parts2 = _sc_conv2()(kqvs2.reshape(-1), e_ch[5], ei_flat)
    out = _final_stage(parts2.reshape(NW, NPAD), kqvs2)
    return out[0, :N].reshape(N, 1)


# speedup vs baseline: 67.6039x; 1.0440x over previous
import functools

import jax
import jax.numpy as jnp
from jax import lax
from jax.experimental import pallas as pl
from jax.experimental.pallas import tpu as pltpu
from jax.experimental.pallas import tpu_sc as plsc

N = 10000
E = 320000
D = 128
ED = 16
H1 = 5
H2 = 1
G = 64

NPAD = 10240
NW = 32
EW = E // NW
LANES = 16
HIGHEST = lax.Precision.HIGHEST


def _tc_node_proj(w_ref, b_ref, x_ref, out_ref):
    out = lax.dot_general(w_ref[...], x_ref[...], (((0,), (1,)), ((), ())),
                          preferred_element_type=jnp.float32, precision=HIGHEST)
    out_ref[...] = out + b_ref[...]


def _node_proj(x_pad, Wcat, bcat):
    return pl.pallas_call(
        _tc_node_proj,
        out_shape=jax.ShapeDtypeStruct((24, NPAD), jnp.float32),
    )(Wcat, bcat, x_pad)


BLK_E = 12800


def _tc_edge_proj(w_ref, b_ref, ea_ref, *out_refs):
    out = lax.dot_general(w_ref[...], ea_ref[...], (((0,), (0,)), ((), ())),
                          preferred_element_type=jnp.float32, precision=HIGHEST)
    out = out + b_ref[...]
    i = pl.program_id(0)
    for c, ref in enumerate(out_refs):
        ref[pl.ds(i * BLK_E, BLK_E)] = out[c]


def _edge_proj(Wecat, becat, ea_t):
    grid = (E // BLK_E,)
    return pl.pallas_call(
        _tc_edge_proj,
        grid=grid,
        in_specs=[
            pl.BlockSpec((ED, 6), lambda i: (0, 0)),
            pl.BlockSpec((6, 1), lambda i: (0, 0)),
            pl.BlockSpec((ED, BLK_E), lambda i: (0, i)),
        ],
        out_specs=[pl.BlockSpec((E,), lambda i: (0,))] * 6,
        out_shape=[jax.ShapeDtypeStruct((E,), jnp.float32)] * 6,
    )(Wecat, becat, ea_t)


@functools.lru_cache(maxsize=None)
def _sc_mesh():
    return plsc.VectorSubcoreMesh(core_axis_name="c", subcore_axis_name="s",
                                  num_cores=2, num_subcores=16)


def _sc_edge_pass(nch, ch_rows, e_list, kqv_hbm, ei_hbm, out_hbm,
                  idx_v, ktab0, qtab0, vtab0, e_v0,
                  ktab1, qtab1, vtab1, e_v1, agg,
                  sem_idx, sem_a, sem_b):
    wid = lax.axis_index("s") * 2 + lax.axis_index("c")
    base = wid * EW
    cp_i = pltpu.async_copy(ei_hbm.at[pl.ds(base, EW)], idx_v, sem_idx)
    kr, qr, vr = ch_rows
    sems = (sem_a, sem_b)
    bufs = ((ktab0, qtab0, vtab0, e_v0), (ktab1, qtab1, vtab1, e_v1))

    def start_tables(c, b):
        kt, qt, vt, ev = bufs[b]
        return [
            pltpu.async_copy(kqv_hbm.at[pl.ds((kr + c) * NPAD, NPAD)],
                             kt, sems[b]),
            pltpu.async_copy(kqv_hbm.at[pl.ds((qr + c) * NPAD, NPAD)],
                             qt, sems[b]),
            pltpu.async_copy(kqv_hbm.at[pl.ds((vr + c) * NPAD, NPAD)],
                             vt, sems[b]),
            pltpu.async_copy(e_list[c].at[pl.ds(base, EW)], ev, sems[b]),
        ]

    pending = start_tables(0, 0)
    cp_i.wait()
    zero = jnp.zeros((LANES,), jnp.float32)
    for c in range(nch):
        b = c & 1
        for cp in pending:
            cp.wait()
        if c + 1 < nch:
            pending = start_tables(c + 1, 1 - b)

        def zbody(i, _):
            for u in range(8):
                agg[pl.ds((i * 8 + u) * LANES, LANES)] = zero
            return 0
        lax.fori_loop(0, NPAD // (8 * LANES), zbody, 0)

        kt, qt, vt, evb = bufs[b]

        @plsc.parallel_loop(0, EW // LANES, 1, unroll=10)
        def _(i):
            off = i * LANES
            pk = idx_v[pl.ds(off, LANES)]
            didx = pk & 0xFFFF
            sidx = lax.shift_right_logical(pk, 16)
            kd = plsc.load_gather(kt, [didx])
            qs = plsc.load_gather(qt, [sidx])
            vs = plsc.load_gather(vt, [sidx])
            ev = evb[pl.ds(off, LANES)]
            z = kd + qs + ev
            g = 1.0 / (1.0 + jnp.exp(-z))
            plsc.addupdate_scatter(agg, [didx], g * vs)

        pltpu.sync_copy(agg, out_hbm.at[pl.ds((c * NW + wid) * NPAD, NPAD)])


def _sc_scratch():
    return [
        pltpu.VMEM((EW,), jnp.int32),
        pltpu.VMEM((NPAD,), jnp.float32),
        pltpu.VMEM((NPAD,), jnp.float32),
        pltpu.VMEM((NPAD,), jnp.float32),
        pltpu.VMEM((EW,), jnp.float32),
        pltpu.VMEM((NPAD,), jnp.float32),
        pltpu.VMEM((NPAD,), jnp.float32),
        pltpu.VMEM((NPAD,), jnp.float32),
        pltpu.VMEM((EW,), jnp.float32),
        pltpu.VMEM((NPAD,), jnp.float32),
        pltpu.SemaphoreType.DMA,
        pltpu.SemaphoreType.DMA,
        pltpu.SemaphoreType.DMA,
    ]


@functools.lru_cache(maxsize=None)
def _sc_conv1():
    @functools.partial(
        pl.kernel,
        out_type=jax.ShapeDtypeStruct((H1 * NW * NPAD,), jnp.float32),
        mesh=_sc_mesh(),
        scratch_types=_sc_scratch(),
        compiler_params=pltpu.CompilerParams(needs_layout_passes=False),
    )
    def body(kqv_hbm, e0, e1, e2, e3, e4, ei_hbm, out_hbm, *scratch):
        _sc_edge_pass(H1, (0, 5, 10), (e0, e1, e2, e3, e4), kqv_hbm,
                      ei_hbm, out_hbm, *scratch)
    return body


@functools.lru_cache(maxsize=None)
def _sc_conv2():
    @functools.partial(
        pl.kernel,
        out_type=jax.ShapeDtypeStruct((NW * NPAD,), jnp.float32),
        mesh=_sc_mesh(),
        scratch_types=_sc_scratch(),
        compiler_params=pltpu.CompilerParams(needs_layout_passes=False),
    )
    def body(kqv_hbm, e5, ei_hbm, out_hbm, *scratch):
        _sc_edge_pass(1, (0, 1, 2), (e5,), kqv_hbm, ei_hbm, out_hbm,
                      *scratch)
    return body


def _tc_norm(parts_ref, kqv_ref, batch_ref, gw_ref, gb_ref, gms_ref,
             w2_ref, b2_ref, out_ref):
    rows = parts_ref[...]
    hs = []
    for c in range(H1):
        agg_c = jnp.sum(rows[c * NW:(c + 1) * NW, :], axis=0, keepdims=True)
        hs.append(agg_c + kqv_ref[15 + c:16 + c, :])
    h = jnp.concatenate(hs, axis=0)

    bcol = batch_ref[...]
    seg = lax.broadcasted_iota(jnp.int32, (NPAD, G), 1)
    m = jnp.where(bcol == seg, 1.0, 0.0)
    stack11 = jnp.concatenate([h, h * h, jnp.ones((1, NPAD), jnp.float32)],
                              axis=0)
    s11 = lax.dot_general(stack11, m, (((1,), (0,)), ((), ())),
                          preferred_element_type=jnp.float32, precision=HIGHEST)
    cnt = jnp.maximum(s11[2 * H1:2 * H1 + 1], 1.0)
    mean = s11[0:H1] / cnt
    msq = s11[H1:2 * H1] / cnt
    gms = gms_ref[...]
    var = msq - (2.0 * gms - gms * gms) * mean * mean
    std = jnp.sqrt(var + 1e-5)
    a = gw_ref[...] / std
    bco = gb_ref[...] - gw_ref[...] * gms * mean / std
    ab = jnp.concatenate([a, bco], axis=0)
    ab_x = lax.dot_general(ab, m, (((1,), (1,)), ((), ())),
                           preferred_element_type=jnp.float32, precision=HIGHEST)
    normed = ab_x[0:H1] * h + ab_x[H1:2 * H1]
    h2 = jnp.maximum(normed, 0.0)
    rows4 = lax.dot_general(w2_ref[...], h2, (((0,), (0,)), ((), ())),
                            preferred_element_type=jnp.float32, precision=HIGHEST)
    out_ref[...] = rows4 + b2_ref[...]


def _norm_stage(parts1_2d, kqv_cm, batch2d, gw, gb, gms, W2cat, b2cat):
    return pl.pallas_call(
        _tc_norm,
        out_shape=jax.ShapeDtypeStruct((4, NPAD), jnp.float32),
    )(parts1_2d, kqv_cm, batch2d, gw, gb, gms, W2cat, b2cat)


def _tc_final(parts_ref, kqvs_ref, out_ref):
    s = jnp.sum(parts_ref[...], axis=0, keepdims=True)
    z = s + kqvs_ref[3:4, :]
    out_ref[...] = 1.0 / (1.0 + jnp.exp(-z))


def _final_stage(parts2, kqvs2):
    return pl.pallas_call(
        _tc_final,
        out_shape=jax.ShapeDtypeStruct((1, NPAD), jnp.float32),
    )(parts2, kqvs2)


def kernel(x, edge_index, edge_attr, batch_idx, Wk1, bk1, Wq1, bq1, Wv1, bv1,
           We1, be1, Ws1, b1, gw, gb, gms, Wk2, bk2, Wq2, bq2, Wv2, bv2,
           We2, be2, Ws2, b2):
    x_pad = jnp.pad(x, ((0, NPAD - N), (0, 0)))
    Wcat = jnp.concatenate(
        [Wk1, Wq1, Wv1, Ws1, jnp.zeros((D, 4), jnp.float32)], axis=1)
    bcat = jnp.concatenate(
        [bk1, bq1, bv1, b1, jnp.zeros((4,), jnp.float32)])[:, None]
    kqv_cm = _node_proj(x_pad, Wcat, bcat)

    Wecat = jnp.concatenate([We1, We2], axis=1)
    becat = jnp.concatenate([be1, be2])[:, None]
    e_ch = _edge_proj(Wecat, becat, edge_attr.T)

    kqv_flat = kqv_cm.reshape(-1)
    ei_flat = (edge_index[0] << 16) | edge_index[1]
    parts1 = _sc_conv1()(kqv_flat, e_ch[0], e_ch[1], e_ch[2], e_ch[3],
                         e_ch[4], ei_flat)

    batch2d = jnp.pad(batch_idx, (0, NPAD - N), constant_values=G)[:, None]
    W2cat = jnp.concatenate([Wk2, Wq2, Wv2, Ws2], axis=1)
    b2cat = jnp.concatenate([bk2, bq2, bv2, b2])[:, None]
    kqvs2 = _norm_stage(parts1.reshape(H1 * NW, NPAD), kqv_cm, batch2d,
                        gw[:, None], gb[:, None], gms[:, None], W2cat, b2cat)

    parts2 = _sc_conv2()(kqvs2.reshape(-1), e_ch[5], ei_flat)
    out = _final_stage(parts2.reshape(NW, NPAD), kqvs2)
    return out[0, :N].reshape(N, 1)
